# baseline (speedup 1.0000x reference)
import jax
import jax.numpy as jnp
from jax import lax
from jax.experimental import pallas as pl
from jax.experimental.pallas import tpu as pltpu


def _tc_gather(N, V, D):
    def body(tok_smem, table_hbm, out_hbm, sem):
        def it(i, c):
            idx = tok_smem[i]
            pltpu.make_async_copy(
                table_hbm.at[pl.ds(idx, 1)], out_hbm.at[pl.ds(i, 1)], sem
            ).start()
            return c

        lax.fori_loop(0, N, it, 0)

        def wt(i, c):
            pltpu.make_async_copy(
                table_hbm.at[pl.ds(0, 1)], out_hbm.at[pl.ds(0, 1)], sem
            ).wait()
            return c

        lax.fori_loop(0, N, wt, 0)

    return pl.pallas_call(
        body,
        in_specs=[
            pl.BlockSpec(memory_space=pltpu.SMEM),
            pl.BlockSpec(memory_space=pl.ANY),
        ],
        out_specs=pl.BlockSpec(memory_space=pl.ANY),
        out_shape=jax.ShapeDtypeStruct((N, D), jnp.float32),
        scratch_shapes=[pltpu.SemaphoreType.DMA],
    )


def kernel(tokens, W_E):
    B, S = tokens.shape
    V, D = W_E.shape
    N = B * S
    out = _tc_gather(N, V, D)(tokens.reshape(N).astype(jnp.int32), W_E)
    return out.reshape(B, S, D)


# speedup vs baseline: 31.1632x; 31.1632x over previous
import functools

import jax
import jax.numpy as jnp
from jax import lax
from jax.experimental import pallas as pl
from jax.experimental.pallas import tpu as pltpu
from jax.experimental.pallas import tpu_sc as plsc

_NBUF = 8
_CHUNK = 8


def _build_embed(B, S, V, D, n_per_w):
    mesh = plsc.VectorSubcoreMesh(core_axis_name="c", subcore_axis_name="s")
    info = plsc.get_sparse_core_info()
    nc = info.num_cores
    n_chunks = n_per_w // _CHUNK
    n_outer = n_chunks // _NBUF
    w_per_row = S // n_per_w

    @functools.partial(
        pl.kernel,
        mesh=mesh,
        out_type=jax.ShapeDtypeStruct((B, S, D), jnp.float32),
        scratch_types=[
            pltpu.VMEM((n_per_w,), jnp.int32),
            pltpu.VMEM((_NBUF, _CHUNK, D), jnp.float32),
            pltpu.SemaphoreType.DMA((_NBUF,)),
            pltpu.SemaphoreType.DMA((_NBUF,)),
        ],
    )
    def embed(idx_hbm, table_hbm, out_hbm, idx_v, rows_v, gsem, ssem):
        wid = lax.axis_index("s") * nc + lax.axis_index("c")
        row = wid // w_per_row
        col = (wid % w_per_row) * n_per_w
        pltpu.sync_copy(idx_hbm.at[row, pl.ds(col, n_per_w)], idx_v)

        def gather(c, b):
            return pltpu.make_async_copy(
                table_hbm.at[idx_v.at[pl.ds(c * _CHUNK, _CHUNK)]],
                rows_v.at[b],
                gsem.at[b],
            )

        def scatter(c, b):
            return pltpu.make_async_copy(
                rows_v.at[b],
                out_hbm.at[row, pl.ds(col + c * _CHUNK, _CHUNK)],
                ssem.at[b],
            )

        for b in range(_NBUF):
            gather(b, b).start()

        def outer(o, carry):
            c0 = o * _NBUF
            for b in range(_NBUF):
                gather(c0 + b, b).wait()
                scatter(c0 + b, b).start()
            for b in range(_NBUF):
                scatter(c0 + b, b).wait()
                gather(c0 + _NBUF + b, b).start()
            return carry

        lax.fori_loop(0, n_outer - 1, outer, 0)

        c0 = (n_outer - 1) * _NBUF
        for b in range(_NBUF):
            gather(c0 + b, b).wait()
            scatter(c0 + b, b).start()
        for b in range(_NBUF):
            scatter(c0 + b, b).wait()

    return embed


def kernel(tokens, W_E):
    B, S = tokens.shape
    V, D = W_E.shape
    N = B * S
    info = plsc.get_sparse_core_info()
    nw = info.num_cores * info.num_subcores
    n_per_w = N // nw
    return _build_embed(B, S, V, D, n_per_w)(tokens.astype(jnp.int32), W_E)
